# baseline (speedup 1.0000x reference)
import functools

import jax
import jax.numpy as jnp
from jax import lax
from jax.experimental import pallas as pl
from jax.experimental.pallas import tpu as pltpu
from jax.experimental.pallas import tpu_sc as plsc

DIM = 64
NUM_WORKERS = 32
GATHER_CHUNK = 128
BLOCK = 4096
BATCH = 16384
HALF = BATCH // 2


def _sc_gather_packed(table, t_half):
    n = t_half.shape[0]
    b_per_w = n // NUM_WORKERS
    nchunks = b_per_w // GATHER_CHUNK
    mesh = plsc.VectorSubcoreMesh(core_axis_name="c", subcore_axis_name="s")

    @functools.partial(
        pl.kernel,
        mesh=mesh,
        compiler_params=pltpu.CompilerParams(
            use_tc_tiling_on_sc=False, skip_device_barrier=True
        ),
        out_type=jax.ShapeDtypeStruct((n // 2, 2 * DIM), jnp.float32),
        scratch_types=[
            pltpu.VMEM((b_per_w,), jnp.int32),
            pltpu.VMEM((b_per_w, DIM), jnp.float32),
            pltpu.SemaphoreType.DMA,
        ],
    )
    def gather_kernel(tab_hbm, t_hbm, out_hbm, idx_v, rows_v, sem_g):
        wid = lax.axis_index("s") * 2 + lax.axis_index("c")
        base = wid * b_per_w
        rowbase = 1024 * (base // 2048) + base % 1024
        colbase = DIM * ((base % 2048) // 1024)
        pltpu.sync_copy(t_hbm.at[pl.ds(base, b_per_w)], idx_v)
        gathers = [
            pltpu.async_copy(
                tab_hbm.at[idx_v.at[pl.ds(j * GATHER_CHUNK, GATHER_CHUNK)]],
                rows_v.at[pl.ds(j * GATHER_CHUNK, GATHER_CHUNK)],
                sem_g,
            )
            for j in range(nchunks)
        ]
        for g in gathers:
            g.wait()
        pltpu.sync_copy(
            rows_v,
            out_hbm.at[pl.ds(rowbase, b_per_w), pl.ds(colbase, DIM)],
        )

    return gather_kernel(table, t_half)


def _fused_body(xt_ref, g_ref, w_ref, b_ref, o_ref):
    xw = lax.dot_general(
        w_ref[:, :DIM], xt_ref[...],
        (((1,), (0,)), ((), ())),
        preferred_element_type=jnp.float32,
    )
    bias = b_ref[...]
    for sub in range(BLOCK // 2048):
        yev = lax.dot_general(
            w_ref[:, DIM:], g_ref[sub * 1024:(sub + 1) * 1024, :DIM],
            (((1,), (1,)), ((), ())),
            preferred_element_type=jnp.float32,
        )
        yod = lax.dot_general(
            w_ref[:, DIM:], g_ref[sub * 1024:(sub + 1) * 1024, DIM:],
            (((1,), (1,)), ((), ())),
            preferred_element_type=jnp.float32,
        )
        c0 = sub * 2048
        o_ref[:, c0:c0 + 1024] = xw[:, c0:c0 + 1024] + yev + bias
        o_ref[:, c0 + 1024:c0 + 2048] = xw[:, c0 + 1024:c0 + 2048] + yod + bias


def _fused_body_alias(xt_ref, g_ref, w_ref, b_ref, prev_ref, o_ref):
    del prev_ref
    _fused_body(xt_ref, g_ref, w_ref, b_ref, o_ref)


def _fused_matmul_half(xt, g128, W, b_col, col0, prev=None):
    grid = HALF // BLOCK
    blk0 = col0 // BLOCK
    in_specs = [
        pl.BlockSpec((DIM, BLOCK), lambda i: (0, i + blk0)),
        pl.BlockSpec((BLOCK // 2, 2 * DIM), lambda i: (i, 0)),
        pl.BlockSpec((DIM, 2 * DIM), lambda i: (0, 0)),
        pl.BlockSpec((DIM, 1), lambda i: (0, 0)),
    ]
    args = [xt, g128, W, b_col]
    body = _fused_body
    kwargs = {}
    if prev is not None:
        in_specs.append(pl.BlockSpec(memory_space=pl.ANY))
        args.append(prev)
        body = _fused_body_alias
        kwargs["input_output_aliases"] = {4: 0}
    return pl.pallas_call(
        body,
        grid=(grid,),
        in_specs=in_specs,
        out_specs=pl.BlockSpec((DIM, BLOCK), lambda i: (0, i + blk0)),
        out_shape=jax.ShapeDtypeStruct((DIM, BATCH), jnp.float32),
        **kwargs,
    )(*args)


def kernel(x, t, time_embed, W, b):
    t32 = t.astype(jnp.int32)
    xt = jnp.transpose(x)
    b_col = b.reshape(DIM, 1)
    g_a = _sc_gather_packed(time_embed, t32[:HALF])
    g_b = _sc_gather_packed(time_embed, t32[HALF:])
    out_a = _fused_matmul_half(xt, g_a, W, b_col, 0)
    out_t = _fused_matmul_half(xt, g_b, W, b_col, HALF, prev=out_a)
    return jnp.transpose(out_t)


# speedup vs baseline: 1.0753x; 1.0753x over previous
import functools

import jax
import jax.numpy as jnp
from jax import lax
from jax.experimental import pallas as pl
from jax.experimental.pallas import tpu as pltpu
from jax.experimental.pallas import tpu_sc as plsc

DIM = 64
NUM_WORKERS = 32
GATHER_CHUNK = 128
BLOCK = 4096


def _sc_gather_packed(table, t):
    batch = t.shape[0]
    b_per_w = batch // NUM_WORKERS
    nchunks = b_per_w // GATHER_CHUNK
    mesh = plsc.VectorSubcoreMesh(core_axis_name="c", subcore_axis_name="s")

    @functools.partial(
        pl.kernel,
        mesh=mesh,
        compiler_params=pltpu.CompilerParams(
            use_tc_tiling_on_sc=False, skip_device_barrier=True
        ),
        out_type=jax.ShapeDtypeStruct((batch // 2, 2 * DIM), jnp.float32),
        scratch_types=[
            pltpu.VMEM((b_per_w,), jnp.int32),
            pltpu.VMEM((b_per_w, DIM), jnp.float32),
            pltpu.SemaphoreType.DMA,
            pltpu.SemaphoreType.DMA,
        ],
    )
    def gather_kernel(tab_hbm, t_hbm, out_hbm, idx_v, rows_v, sem_i, sem_g):
        wid = lax.axis_index("s") * 2 + lax.axis_index("c")
        base = wid * b_per_w
        rowbase = 1024 * (base // 2048) + base % 1024
        colbase = DIM * ((base % 2048) // 1024)
        idx_copies = [
            pltpu.async_copy(
                t_hbm.at[pl.ds(base + j * GATHER_CHUNK, GATHER_CHUNK)],
                idx_v.at[pl.ds(j * GATHER_CHUNK, GATHER_CHUNK)],
                sem_i,
            )
            for j in range(nchunks)
        ]
        gathers = []
        for j in range(nchunks):
            idx_copies[j].wait()
            gathers.append(
                pltpu.async_copy(
                    tab_hbm.at[idx_v.at[pl.ds(j * GATHER_CHUNK, GATHER_CHUNK)]],
                    rows_v.at[pl.ds(j * GATHER_CHUNK, GATHER_CHUNK)],
                    sem_g,
                )
            )
        for g in gathers:
            g.wait()
        pltpu.sync_copy(
            rows_v,
            out_hbm.at[pl.ds(rowbase, b_per_w), pl.ds(colbase, DIM)],
        )

    return gather_kernel(table, t)


def _fused_body(xt_ref, g_ref, w_ref, b_ref, o_ref):
    xw = lax.dot_general(
        w_ref[:, :DIM], xt_ref[...],
        (((1,), (0,)), ((), ())),
        preferred_element_type=jnp.float32,
    )
    bias = b_ref[...]
    for sub in range(BLOCK // 2048):
        yev = lax.dot_general(
            w_ref[:, DIM:], g_ref[sub * 1024:(sub + 1) * 1024, :DIM],
            (((1,), (1,)), ((), ())),
            preferred_element_type=jnp.float32,
        )
        yod = lax.dot_general(
            w_ref[:, DIM:], g_ref[sub * 1024:(sub + 1) * 1024, DIM:],
            (((1,), (1,)), ((), ())),
            preferred_element_type=jnp.float32,
        )
        c0 = sub * 2048
        o_ref[:, c0:c0 + 1024] = xw[:, c0:c0 + 1024] + yev + bias
        o_ref[:, c0 + 1024:c0 + 2048] = xw[:, c0 + 1024:c0 + 2048] + yod + bias


def _fused_matmul_t(xt, g128, W, b_col):
    batch = xt.shape[1]
    grid = batch // BLOCK
    return pl.pallas_call(
        _fused_body,
        grid=(grid,),
        in_specs=[
            pl.BlockSpec((DIM, BLOCK), lambda i: (0, i)),
            pl.BlockSpec((BLOCK // 2, 2 * DIM), lambda i: (i, 0)),
            pl.BlockSpec((DIM, 2 * DIM), lambda i: (0, 0)),
            pl.BlockSpec((DIM, 1), lambda i: (0, 0)),
        ],
        out_specs=pl.BlockSpec((DIM, BLOCK), lambda i: (0, i)),
        out_shape=jax.ShapeDtypeStruct((DIM, batch), jnp.float32),
    )(xt, g128, W, b_col)


def kernel(x, t, time_embed, W, b):
    g128 = _sc_gather_packed(time_embed, t.astype(jnp.int32))
    out_t = _fused_matmul_t(
        jnp.transpose(x), g128, W, b.reshape(DIM, 1)
    )
    return jnp.transpose(out_t)


# speedup vs baseline: 1.1079x; 1.0303x over previous
import functools

import jax
import jax.numpy as jnp
from jax import lax
from jax.experimental import pallas as pl
from jax.experimental.pallas import tpu as pltpu
from jax.experimental.pallas import tpu_sc as plsc

DIM = 64
NUM_CORES = 1
NUM_WORKERS = NUM_CORES * 16
GATHER_CHUNK = 128
BLOCK = 4096


def _sc_gather_packed(table, t):
    batch = t.shape[0]
    b_per_w = batch // NUM_WORKERS
    nchunks = b_per_w // GATHER_CHUNK
    mesh = plsc.VectorSubcoreMesh(
        core_axis_name="c", subcore_axis_name="s", num_cores=NUM_CORES
    )

    @functools.partial(
        pl.kernel,
        mesh=mesh,
        compiler_params=pltpu.CompilerParams(
            use_tc_tiling_on_sc=False, skip_device_barrier=True
        ),
        out_type=jax.ShapeDtypeStruct((batch // 2, 2 * DIM), jnp.float32),
        scratch_types=[
            pltpu.VMEM((b_per_w,), jnp.int32),
            pltpu.VMEM((b_per_w, DIM), jnp.float32),
            pltpu.SemaphoreType.DMA,
            pltpu.SemaphoreType.DMA,
        ],
    )
    def gather_kernel(tab_hbm, t_hbm, out_hbm, idx_v, rows_v, sem_i, sem_g):
        wid = lax.axis_index("s") * NUM_CORES + lax.axis_index("c")
        base = wid * b_per_w
        rowbase = 1024 * (base // 2048) + base % 1024
        colbase = DIM * ((base % 2048) // 1024)
        idx_copies = [
            pltpu.async_copy(
                t_hbm.at[pl.ds(base + j * GATHER_CHUNK, GATHER_CHUNK)],
                idx_v.at[pl.ds(j * GATHER_CHUNK, GATHER_CHUNK)],
                sem_i,
            )
            for j in range(nchunks)
        ]
        gathers = []
        for j in range(nchunks):
            idx_copies[j].wait()
            gathers.append(
                pltpu.async_copy(
                    tab_hbm.at[idx_v.at[pl.ds(j * GATHER_CHUNK, GATHER_CHUNK)]],
                    rows_v.at[pl.ds(j * GATHER_CHUNK, GATHER_CHUNK)],
                    sem_g,
                )
            )
        for g in gathers:
            g.wait()
        pltpu.sync_copy(
            rows_v,
            out_hbm.at[pl.ds(rowbase, b_per_w), pl.ds(colbase, DIM)],
        )

    return gather_kernel(table, t)


def _fused_body(xt_ref, g_ref, w_ref, b_ref, o_ref):
    xw = lax.dot_general(
        w_ref[:, :DIM], xt_ref[...],
        (((1,), (0,)), ((), ())),
        preferred_element_type=jnp.float32,
    )
    bias = b_ref[...]
    for sub in range(BLOCK // 2048):
        yev = lax.dot_general(
            w_ref[:, DIM:], g_ref[sub * 1024:(sub + 1) * 1024, :DIM],
            (((1,), (1,)), ((), ())),
            preferred_element_type=jnp.float32,
        )
        yod = lax.dot_general(
            w_ref[:, DIM:], g_ref[sub * 1024:(sub + 1) * 1024, DIM:],
            (((1,), (1,)), ((), ())),
            preferred_element_type=jnp.float32,
        )
        c0 = sub * 2048
        o_ref[:, c0:c0 + 1024] = xw[:, c0:c0 + 1024] + yev + bias
        o_ref[:, c0 + 1024:c0 + 2048] = xw[:, c0 + 1024:c0 + 2048] + yod + bias


def _fused_matmul_t(xt, g128, W, b_col):
    batch = xt.shape[1]
    grid = batch // BLOCK
    return pl.pallas_call(
        _fused_body,
        grid=(grid,),
        in_specs=[
            pl.BlockSpec((DIM, BLOCK), lambda i: (0, i)),
            pl.BlockSpec((BLOCK // 2, 2 * DIM), lambda i: (i, 0)),
            pl.BlockSpec((DIM, 2 * DIM), lambda i: (0, 0)),
            pl.BlockSpec((DIM, 1), lambda i: (0, 0)),
        ],
        out_specs=pl.BlockSpec((DIM, BLOCK), lambda i: (0, i)),
        out_shape=jax.ShapeDtypeStruct((DIM, batch), jnp.float32),
    )(xt, g128, W, b_col)


def kernel(x, t, time_embed, W, b):
    g128 = _sc_gather_packed(time_embed, t.astype(jnp.int32))
    out_t = _fused_matmul_t(
        jnp.transpose(x), g128, W, b.reshape(DIM, 1)
    )
    return jnp.transpose(out_t)


# speedup vs baseline: 1.1087x; 1.0008x over previous
import functools

import jax
import jax.numpy as jnp
from jax import lax
from jax.experimental import pallas as pl
from jax.experimental.pallas import tpu as pltpu
from jax.experimental.pallas import tpu_sc as plsc

DIM = 64
NUM_CORES = 1
NUM_WORKERS = NUM_CORES * 16
GATHER_CHUNK = 128
BLOCK = 4096


def _sc_gather_packed(table, t):
    batch = t.shape[0]
    b_per_w = batch // NUM_WORKERS
    nchunks = b_per_w // GATHER_CHUNK
    mesh = plsc.VectorSubcoreMesh(
        core_axis_name="c", subcore_axis_name="s", num_cores=NUM_CORES
    )

    @functools.partial(
        pl.kernel,
        mesh=mesh,
        compiler_params=pltpu.CompilerParams(
            use_tc_tiling_on_sc=False, skip_device_barrier=True
        ),
        out_type=jax.ShapeDtypeStruct((batch // 2, 2 * DIM), jnp.float32),
        scratch_types=[
            pltpu.VMEM((b_per_w,), jnp.int32),
            pltpu.VMEM((b_per_w, DIM), jnp.float32),
            pltpu.SemaphoreType.DMA,
            pltpu.SemaphoreType.DMA,
            pltpu.SemaphoreType.DMA,
        ],
    )
    def gather_kernel(tab_hbm, t_hbm, out_hbm, idx_v, rows_v, sem_i, sem_g, sem_w):
        wid = lax.axis_index("s") * NUM_CORES + lax.axis_index("c")
        base = wid * b_per_w
        rowbase = 1024 * (base // 2048) + base % 1024
        colbase = DIM * ((base % 2048) // 1024)
        idx_copies = [
            pltpu.async_copy(
                t_hbm.at[pl.ds(base + j * GATHER_CHUNK, GATHER_CHUNK)],
                idx_v.at[pl.ds(j * GATHER_CHUNK, GATHER_CHUNK)],
                sem_i,
            )
            for j in range(nchunks)
        ]
        gathers = []
        for j in range(nchunks):
            idx_copies[j].wait()
            gathers.append(
                pltpu.async_copy(
                    tab_hbm.at[idx_v.at[pl.ds(j * GATHER_CHUNK, GATHER_CHUNK)]],
                    rows_v.at[pl.ds(j * GATHER_CHUNK, GATHER_CHUNK)],
                    sem_g,
                )
            )
        half = b_per_w // 2
        for j in range(nchunks // 2):
            gathers[j].wait()
        w0 = pltpu.async_copy(
            rows_v.at[pl.ds(0, half)],
            out_hbm.at[pl.ds(rowbase, half), pl.ds(colbase, DIM)],
            sem_w,
        )
        for j in range(nchunks // 2, nchunks):
            gathers[j].wait()
        w1 = pltpu.async_copy(
            rows_v.at[pl.ds(half, half)],
            out_hbm.at[pl.ds(rowbase + half, half), pl.ds(colbase, DIM)],
            sem_w,
        )
        w0.wait()
        w1.wait()

    return gather_kernel(table, t)


def _fused_body(xt_ref, g_ref, w_ref, b_ref, o_ref):
    xw = lax.dot_general(
        w_ref[:, :DIM], xt_ref[...],
        (((1,), (0,)), ((), ())),
        preferred_element_type=jnp.float32,
    )
    bias = b_ref[...]
    for sub in range(BLOCK // 2048):
        yev = lax.dot_general(
            w_ref[:, DIM:], g_ref[sub * 1024:(sub + 1) * 1024, :DIM],
            (((1,), (1,)), ((), ())),
            preferred_element_type=jnp.float32,
        )
        yod = lax.dot_general(
            w_ref[:, DIM:], g_ref[sub * 1024:(sub + 1) * 1024, DIM:],
            (((1,), (1,)), ((), ())),
            preferred_element_type=jnp.float32,
        )
        c0 = sub * 2048
        o_ref[:, c0:c0 + 1024] = xw[:, c0:c0 + 1024] + yev + bias
        o_ref[:, c0 + 1024:c0 + 2048] = xw[:, c0 + 1024:c0 + 2048] + yod + bias


def _fused_matmul_t(xt, g128, W, b_col):
    batch = xt.shape[1]
    grid = batch // BLOCK
    return pl.pallas_call(
        _fused_body,
        grid=(grid,),
        in_specs=[
            pl.BlockSpec((DIM, BLOCK), lambda i: (0, i)),
            pl.BlockSpec((BLOCK // 2, 2 * DIM), lambda i: (i, 0)),
            pl.BlockSpec((DIM, 2 * DIM), lambda i: (0, 0)),
            pl.BlockSpec((DIM, 1), lambda i: (0, 0)),
        ],
        out_specs=pl.BlockSpec((DIM, BLOCK), lambda i: (0, i)),
        out_shape=jax.ShapeDtypeStruct((DIM, batch), jnp.float32),
    )(xt, g128, W, b_col)


def kernel(x, t, time_embed, W, b):
    g128 = _sc_gather_packed(time_embed, t.astype(jnp.int32))
    out_t = _fused_matmul_t(
        jnp.transpose(x), g128, W, b.reshape(DIM, 1)
    )
    return jnp.transpose(out_t)
